# Initial kernel scaffold; baseline (speedup 1.0000x reference)
import jax
import jax.numpy as jnp
from jax.experimental import pallas as pl


def kernel(x, entity_vec0, entity_vec1, relation_vec0, encode_w0, encode_b0, encode_w1, encode_b1, enc2u_w, enc2u_b, fc_w, fc_b):
    raise NotImplementedError("write your pallas kernel here")


# speedup vs baseline: 47.8220x; 47.8220x over previous
import functools

import jax
import jax.numpy as jnp
from jax import lax
from jax.experimental import pallas as pl
from jax.experimental.pallas import tpu as pltpu

N_ITEM = 25274
DIM = 16
K_NB = 4
BATCH = 64

_SELU_SCALE = 1.0507009873554805
_SELU_ALPHA = 1.6732632423543772


def _selu(v):
    return _SELU_SCALE * jnp.where(v > 0, v, _SELU_ALPHA * (jnp.exp(v) - 1.0))


def _encode_body(x_ref, w0_ref, b0_ref, w1_ref, b1_ref, w2_ref, b2_ref,
                 u_ref, acc_ref, *, kb, n_total, n_steps):
    i = pl.program_id(0)

    @pl.when(i == 0)
    def _():
        acc_ref[...] = jnp.zeros_like(acc_ref)

    xb = x_ref[...]
    wb = w0_ref[...]
    col = i * kb + lax.broadcasted_iota(jnp.int32, (1, kb), 1)
    valid = col < n_total
    xb = jnp.where(valid, xb, 0.0)
    wb = jnp.where(valid, wb, 0.0)
    acc_ref[...] += lax.dot_general(
        xb, wb, (((1,), (1,)), ((), ())), preferred_element_type=jnp.float32)

    @pl.when(i == n_steps - 1)
    def _():
        h = _selu(acc_ref[...] + b0_ref[...])
        h = _selu(lax.dot_general(
            h, w1_ref[...], (((1,), (1,)), ((), ())),
            preferred_element_type=jnp.float32) + b1_ref[...])
        u = lax.dot_general(
            h, w2_ref[...], (((1,), (1,)), ((), ())),
            preferred_element_type=jnp.float32) + b2_ref[...]
        u_ref[...] = u


def _encode(x, w0, b0, w1, b1, w2, b2):
    kb = 2048
    n_steps = pl.cdiv(N_ITEM, kb)
    grid = (n_steps,)
    return pl.pallas_call(
        functools.partial(_encode_body, kb=kb, n_total=N_ITEM,
                          n_steps=n_steps),
        grid=grid,
        in_specs=[
            pl.BlockSpec((BATCH, kb), lambda i: (0, i)),
            pl.BlockSpec((512, kb), lambda i: (0, i)),
            pl.BlockSpec((1, 512), lambda i: (0, 0)),
            pl.BlockSpec((128, 512), lambda i: (0, 0)),
            pl.BlockSpec((1, 128), lambda i: (0, 0)),
            pl.BlockSpec((DIM, 128), lambda i: (0, 0)),
            pl.BlockSpec((1, DIM), lambda i: (0, 0)),
        ],
        out_specs=pl.BlockSpec((BATCH, DIM), lambda i: (0, 0)),
        out_shape=jax.ShapeDtypeStruct((BATCH, DIM), jnp.float32),
        scratch_shapes=[pltpu.VMEM((BATCH, 512), jnp.float32)],
    )(x, w0, b0.reshape(1, 512), w1, b1.reshape(1, 128), w2,
      b2.reshape(1, DIM))


def _agg_body(u_ref, e0t_ref, relt_ref, nbt_ref, fcw_ref, fcb_ref, out_ref):
    u = u_ref[...]
    w_self = fcw_ref[:, :DIM]
    w_nb = fcw_ref[:, DIM:]

    s0 = lax.dot_general(w_self, e0t_ref[...], (((1,), (0,)), ((), ())),
                         preferred_element_type=jnp.float32) + fcb_ref[...]

    sc = [lax.dot_general(u, relt_ref[k], (((1,), (0,)), ((), ())),
                          preferred_element_type=jnp.float32) * (1.0 / DIM)
          for k in range(K_NB)]
    m = jnp.maximum(jnp.maximum(sc[0], sc[1]), jnp.maximum(sc[2], sc[3]))
    e = [jnp.exp(s - m) for s in sc]
    rden = 1.0 / (e[0] + e[1] + e[2] + e[3])

    z = jnp.broadcast_to(s0[None], (BATCH,) + s0.shape)
    for k in range(K_NB):
        nbw = lax.dot_general(w_nb, nbt_ref[k], (((1,), (0,)), ((), ())),
                              preferred_element_type=jnp.float32)
        z = z + (e[k] * rden)[:, None, :] * nbw[None, :, :]
    out = jnp.tanh(z)
    res = jnp.sum(out * u[:, :, None], axis=1)
    out_ref[...] = 1.0 / (1.0 + jnp.exp(-res))


def _aggregate(u, e0t, relt, nbt, fc_w, fc_bt, n_items, nb):
    grid = (pl.cdiv(n_items, nb),)
    return pl.pallas_call(
        _agg_body,
        grid=grid,
        in_specs=[
            pl.BlockSpec((BATCH, DIM), lambda i: (0, 0)),
            pl.BlockSpec((DIM, nb), lambda i: (0, i)),
            pl.BlockSpec((K_NB, DIM, nb), lambda i: (0, 0, i)),
            pl.BlockSpec((K_NB, DIM, nb), lambda i: (0, 0, i)),
            pl.BlockSpec((DIM, 2 * DIM), lambda i: (0, 0)),
            pl.BlockSpec((DIM, 1), lambda i: (0, 0)),
        ],
        out_specs=pl.BlockSpec((BATCH, nb), lambda i: (0, i)),
        out_shape=jax.ShapeDtypeStruct((BATCH, n_items), jnp.float32),
    )(u, e0t, relt, nbt, fc_w, fc_bt)


def kernel(x, entity_vec0, entity_vec1, relation_vec0, encode_w0, encode_b0,
           encode_w1, encode_b1, enc2u_w, enc2u_b, fc_w, fc_b):
    u = _encode(x, encode_w0, encode_b0, encode_w1, encode_b1,
                enc2u_w, enc2u_b)

    e0t = entity_vec0.T
    nbt = entity_vec1.reshape(N_ITEM, K_NB, DIM).transpose(1, 2, 0)
    relt = relation_vec0.reshape(N_ITEM, K_NB, DIM).transpose(1, 2, 0)
    fc_bt = fc_b.reshape(DIM, 1)

    return _aggregate(u, e0t, relt, nbt, fc_w, fc_bt, N_ITEM, 512)
